# baseline (speedup 1.0000x reference)
import functools

import jax
import jax.numpy as jnp
from jax import lax
from jax.experimental import pallas as pl
from jax.experimental.pallas import tpu as pltpu
from jax.experimental.pallas import tpu_sc as plsc

_N_TARGETS = 6
_R = 5
_H = 512
_W = 512
_AH = _H - _R + 1
_AREA = 25.0
_LANE = 16
_NV = _W // _LANE
_NB = 16


def _iota2(shape, dim):
    return jax.lax.broadcasted_iota(jnp.int32, shape, dim)


def _prologue_kernel(hm_ref, aggmax_ref, aggcol_ref, hmmax_ref):
    hm = hm_ref[0]

    acc = hm
    for k in range(1, _R):
        acc = acc + jnp.concatenate(
            [hm[:, k:], jnp.zeros((_H, k), jnp.float32)], axis=1)
    v = acc[0:_AH, :]
    for k in range(1, _R):
        v = v + acc[k:k + _AH, :]
    colia = _iota2((_AH, _W), 1)
    agg = jnp.where(colia < _AH, v / _AREA, -1.0)

    rmax = jnp.max(agg, axis=1)
    argc = jnp.min(
        jnp.where(agg == rmax[:, None], colia, jnp.int32(_W)), axis=1)
    aggmax_ref[0, 0, :] = jnp.concatenate(
        [rmax, jnp.full((_H - _AH,), -1.0, jnp.float32)])
    aggcol_ref[0, 0, :] = jnp.concatenate(
        [argc, jnp.zeros((_H - _AH,), jnp.int32)])
    hmmax_ref[0, 0, :] = jnp.max(hm, axis=1)


def _prologue(hm):
    b = hm.shape[0]
    return pl.pallas_call(
        _prologue_kernel,
        grid=(b,),
        in_specs=[pl.BlockSpec((1, _H, _W), lambda i: (i, 0, 0))],
        out_specs=[
            pl.BlockSpec((1, 1, _W), lambda i: (i, 0, 0)),
            pl.BlockSpec((1, 1, _W), lambda i: (i, 0, 0)),
            pl.BlockSpec((1, 1, _W), lambda i: (i, 0, 0)),
        ],
        out_shape=[
            jax.ShapeDtypeStruct((b, 1, _W), jnp.float32),
            jax.ShapeDtypeStruct((b, 1, _W), jnp.int32),
            jax.ShapeDtypeStruct((b, 1, _W), jnp.float32),
        ],
        compiler_params=pltpu.CompilerParams(
            dimension_semantics=("parallel",),
        ),
    )(hm)


def _gather_row(ref, row, cols):
    rows = jnp.full((_LANE,), 1, jnp.int32) * row
    cc = jnp.minimum(cols, jnp.int32(_W - 1))
    return plsc.load_gather(ref, [rows, cc])


def _zero_win(ref, row, c, lanes):
    rows = jnp.full((_LANE,), 1, jnp.int32) * row
    cc = jnp.minimum(c + lanes, jnp.int32(_W - 1))
    plsc.store_scatter(ref, [rows, cc], jnp.zeros((_LANE,), jnp.float32),
                       mask=lanes < _R)


def _vmax(v):
    return lax.reduce_max(v, (0,))


def _vmin(v):
    return lax.reduce_min(v, (0,))


def _vsum(v):
    return lax.reduce_sum(v, (0,))


_LANES = None


def _scalar_load(ref, idx, lanes):
    base = (idx // _LANE) * _LANE
    v = ref[pl.ds(base, _LANE)]
    zero = jnp.zeros((), v.dtype)
    return _vsum(jnp.where(lanes == idx - base, v, zero))


def _scalar_store(ref, idx, val, lanes):
    base = (idx // _LANE) * _LANE
    v = ref[pl.ds(base, _LANE)]
    ref[pl.ds(base, _LANE)] = jnp.where(lanes == idx - base, val, v)


def _sc_kernel(hm2d, aggmax_h, aggcol_h, hmmax_h, out_h,
               aggmax_v, aggcol_v, hmmax_v, rows_v, hbuf_v, gbuf_v,
               cand_v, wr_v, wc_v, outbuf_v, sem):
    cid = lax.axis_index("c")
    sid = lax.axis_index("s")
    b = sid * 2 + cid
    lanes = lax.iota(jnp.int32, _LANE)

    @pl.when(b < _NB)
    def _():
        pltpu.async_copy(aggmax_h.at[b, 0, :], aggmax_v, sem).wait()
        pltpu.async_copy(aggcol_h.at[b, 0, :], aggcol_v, sem).wait()
        pltpu.async_copy(hmmax_h.at[b, 0, :], hmmax_v, sem).wait()
        for _q in range(8):
            outbuf_v[_q, :] = jnp.zeros((_LANE,), jnp.float32)

        def iteration(i, carry):
            def max_body(k, m):
                return jnp.maximum(m, _vmax(aggmax_v[pl.ds(k * _LANE,
                                                           _LANE)]))
            gmax = lax.fori_loop(0, _NV, max_body, jnp.float32(-2.0))

            def row_body(k, rr):
                v = aggmax_v[pl.ds(k * _LANE, _LANE)]
                idx = jnp.where(v == gmax, k * _LANE + lanes, jnp.int32(_H))
                return jnp.minimum(rr, _vmin(idx))
            r = lax.fori_loop(0, _NV, row_body, jnp.int32(_H))
            c = _scalar_load(aggcol_v, r, lanes)

            r0 = jnp.minimum(jnp.maximum(r - (_R - 1), 0), _H - 13)
            ra = pl.multiple_of(
                jnp.minimum((r0 // 8) * 8, _H - 24), 8)
            loff = r0 - ra
            pltpu.async_copy(hm2d.at[pl.ds(b * _H + ra, 24), :],
                             rows_v, sem).wait()

            def replay(j, _):
                rj = _scalar_load(wr_v, j, lanes)
                cj = _scalar_load(wc_v, j, lanes)

                def zrow(a, _):
                    @pl.when((a >= ra) & (a < ra + 24))
                    def _():
                        _zero_win(rows_v, a - ra, cj, lanes)
                    return 0
                lax.fori_loop(rj, rj + _R, zrow, 0)
                return 0
            lax.fori_loop(0, i, replay, 0)

            def wstats(j, mc):
                m, s = mc
                vv = _gather_row(rows_v, (r - ra) + j, c + lanes)
                m = jnp.maximum(m, _vmax(jnp.where(lanes < _R, vv, -1.0)))
                s = s + _vsum(jnp.where(lanes < _R, vv, 0.0))
                return m, s
            mval, conf = lax.fori_loop(
                0, _R, wstats, (jnp.float32(-1.0), jnp.float32(0.0)))

            def attempt(state):
                found, rh, ch, minrow = state
                cand_v[...] = jnp.full((_LANE,), _H, jnp.int32)

                def collect(k, st):
                    cnt, minr = st
                    v = hmmax_v[pl.ds(k * _LANE, _LANE)]
                    rowid = k * _LANE + lanes
                    m0 = (v >= mval) & (rowid >= minr)

                    def inner(st2):
                        cnt, m = st2
                        mn = _vmin(jnp.where(m, rowid, jnp.int32(_H)))
                        _scalar_store(cand_v, cnt, mn, lanes)
                        return cnt + 1, m & (rowid > mn)

                    def inner_cond(st2):
                        cnt, m = st2
                        return (cnt < _LANE) & (_vsum(
                            m.astype(jnp.int32)) > 0)
                    cnt, _ = lax.while_loop(inner_cond, inner, (cnt, m0))
                    return cnt, minr
                cnt, _ = lax.fori_loop(0, _NV, collect,
                                       (jnp.int32(0), minrow))

                cv = cand_v[...]
                gidx = jnp.where(cv < _H, b * _H + cv, b * _H)
                pltpu.async_copy(hm2d.at[gidx], gbuf_v, sem).wait()

                def scanrow(q, st4):
                    found, rh, ch = st4
                    rq = _scalar_load(cand_v, q, lanes)

                    def do4(st5):
                        found, rh, ch = st5

                        def replay2(j, _):
                            rj = _scalar_load(wr_v, j, lanes)
                            cj = _scalar_load(wc_v, j, lanes)

                            @pl.when((rq >= rj) & (rq < rj + _R))
                            def _():
                                _zero_win(gbuf_v, q, cj, lanes)
                            return 0
                        lax.fori_loop(0, i, replay2, 0)

                        def col_body(k, st6):
                            fnd, chh = st6

                            def do6(st7):
                                fnd, chh = st7
                                v = gbuf_v[q, pl.ds(k * _LANE, _LANE)]
                                f = _vmin(jnp.where(v == mval, lanes,
                                                    jnp.int32(_LANE)))
                                hit = f < _LANE
                                return (fnd | hit,
                                        jnp.where(hit, k * _LANE + f, chh))
                            return lax.cond(fnd, lambda s7: s7, do6, st6)
                        fnd2, ch2 = lax.fori_loop(
                            0, _NV, col_body,
                            (jnp.bool_(False), jnp.int32(0)))
                        rh = jnp.where(fnd2, rq, rh)
                        ch = jnp.where(fnd2, ch2, ch)
                        return found | fnd2, rh, ch
                    ok = jnp.logical_not(found) & (q < cnt) & (rq < _H)
                    return lax.cond(ok, do4, lambda s5: s5, st4)
                found, rh, ch = lax.fori_loop(0, _LANE, scanrow,
                                              (found, rh, ch))
                last = _scalar_load(cand_v, jnp.maximum(cnt - 1, 0), lanes)
                nminrow = jnp.where(cnt > 0, last + 1, jnp.int32(_H))
                return found, rh, ch, nminrow

            def att_cond(state):
                found, _, _, minrow = state
                return jnp.logical_not(found) & (minrow < _H)
            found, rh, ch, _ = lax.while_loop(
                att_cond, attempt,
                (jnp.bool_(False), jnp.int32(0), jnp.int32(0), jnp.int32(0)))

            def zcur(j, _):
                _zero_win(rows_v, (r - ra) + j, c, lanes)
                return 0
            lax.fori_loop(0, _R, zcur, 0)
            _scalar_store(wr_v, i, r, lanes)
            _scalar_store(wc_v, i, c, lanes)

            def hmrow(j, _):
                def mb(k, m):
                    return jnp.maximum(m, _vmax(
                        rows_v[(r - ra) + j, pl.ds(k * _LANE, _LANE)]))
                _scalar_store(
                    hmmax_v, r + j,
                    lax.fori_loop(0, _NV, mb, jnp.float32(-1.0)), lanes)
                return 0
            lax.fori_loop(0, _R, hmrow, 0)

            def hrow(j, _):
                def hcol(k, _):
                    acc = rows_v[loff + j, pl.ds(k * _LANE, _LANE)]
                    for t in range(1, _R):
                        acc = acc + _gather_row(rows_v, loff + j,
                                                k * _LANE + t + lanes)
                    hbuf_v[j, pl.ds(k * _LANE, _LANE)] = acc
                    return 0
                lax.fori_loop(0, _NV, hcol, 0)
                return 0
            lax.fori_loop(0, 13, hrow, 0)

            amin = jnp.maximum(r - (_R - 1), 0)
            amax = jnp.minimum(r + (_R - 1), _AH - 1)

            def arow(a, _):
                la = a - r0

                def acol_max(k, m):
                    acc = hbuf_v[la, pl.ds(k * _LANE, _LANE)]
                    for t in range(1, _R):
                        acc = acc + hbuf_v[la + t, pl.ds(k * _LANE, _LANE)]
                    acc = acc / _AREA
                    colid = k * _LANE + lanes
                    acc = jnp.where(colid < _AH, acc, -1.0)
                    return jnp.maximum(m, _vmax(acc))
                m = lax.fori_loop(0, _NV, acol_max, jnp.float32(-2.0))

                def acol_arg(k, cc):
                    acc = hbuf_v[la, pl.ds(k * _LANE, _LANE)]
                    for t in range(1, _R):
                        acc = acc + hbuf_v[la + t, pl.ds(k * _LANE, _LANE)]
                    acc = acc / _AREA
                    colid = k * _LANE + lanes
                    acc = jnp.where(colid < _AH, acc, -1.0)
                    idx = jnp.where(acc == m, colid, jnp.int32(_W))
                    return jnp.minimum(cc, _vmin(idx))
                cc = lax.fori_loop(0, _NV, acol_arg, jnp.int32(_W))
                _scalar_store(aggmax_v, a, m, lanes)
                _scalar_store(aggcol_v, a, cc, lanes)
                return 0
            lax.fori_loop(amin, amax + 1, arow, 0)

            row8 = outbuf_v[i // 2, :]
            base = (i % 2) * 8
            upd = jnp.where(lanes == base, rh.astype(jnp.float32), row8)
            upd = jnp.where(lanes == base + 1, ch.astype(jnp.float32), upd)
            upd = jnp.where(lanes == base + 2, conf, upd)
            outbuf_v[i // 2, :] = upd
            return carry

        lax.fori_loop(0, _N_TARGETS, iteration, 0)
        pltpu.async_copy(outbuf_v, out_h.at[b], sem).wait()


def _sc_sample(hm2d, aggmax, aggcol, hmmax):
    mesh = plsc.VectorSubcoreMesh(core_axis_name="c", subcore_axis_name="s",
                                  num_cores=2, num_subcores=16)
    fn = functools.partial(
        pl.kernel,
        out_type=jax.ShapeDtypeStruct((_NB, 8, _LANE), jnp.float32),
        mesh=mesh,
        scratch_types=[
            pltpu.VMEM((_H,), jnp.float32),
            pltpu.VMEM((_H,), jnp.int32),
            pltpu.VMEM((_H,), jnp.float32),
            pltpu.VMEM((24, _W), jnp.float32),
            pltpu.VMEM((13, _W), jnp.float32),
            pltpu.VMEM((_LANE, _W), jnp.float32),
            pltpu.VMEM((_LANE,), jnp.int32),
            pltpu.VMEM((_LANE,), jnp.int32),
            pltpu.VMEM((_LANE,), jnp.int32),
            pltpu.VMEM((8, _LANE), jnp.float32),
            pltpu.SemaphoreType.DMA,
        ],
        compiler_params=pltpu.CompilerParams(needs_layout_passes=False),
    )(_sc_kernel)
    return fn(hm2d, aggmax, aggcol, hmmax)


def kernel(heatmap):
    hm = heatmap[:, 0]
    aggmax, aggcol, hmmax = _prologue(hm)
    hm2d = hm.reshape(_NB * _H, _W)
    res = _sc_sample(hm2d, aggmax, aggcol, hmmax)
    res = res.reshape(_NB, 16, 8)[:, :_N_TARGETS, :]
    end_points = res[:, :, 0:2]
    confidences = res[:, :, 2]
    return end_points, confidences


# speedup vs baseline: 1.1828x; 1.1828x over previous
import jax
import jax.numpy as jnp
from jax.experimental import pallas as pl
from jax.experimental.pallas import tpu as pltpu

_N_TARGETS = 6
_R = 5
_H = 512
_W = 512
_AH = _H - _R + 1
_HP = _H + 32


def _iota(shape, dim):
    return jax.lax.broadcasted_iota(jnp.int32, shape, dim)


def _hsum(t):
    acc = t
    n = t.shape[0]
    for k in range(1, _R):
        acc = acc + jnp.concatenate(
            [t[:, k:], jnp.zeros((n, k), jnp.float32)], axis=1)
    return acc


def _rowstats(aggblk, colia):
    m = jnp.max(aggblk, axis=1, keepdims=True)
    cc = jnp.min(jnp.where(aggblk == m, colia, jnp.int32(_W)),
                 axis=1, keepdims=True)
    return m, cc


def _peaks_kernel(hm_ref, out_ref, hm_s, h_s, rmax_s, rcol_s, hmax_s):
    hm0 = hm_ref[0]
    hm_s[...] = hm0
    hmax_s[...] = jnp.max(hm0, axis=1, keepdims=True)

    h0 = _hsum(hm0)
    h_s[0:_H, :] = h0
    h_s[_H:_HP, :] = jnp.zeros((_HP - _H, _W), jnp.float32)

    v = h0[0:_AH, :]
    for k in range(1, _R):
        v = v + h0[k:k + _AH, :]
    colia = _iota((_AH, _W), 1)
    agg0 = jnp.where(colia < _AH, v / float(_R * _R), -1.0)
    m0, c0 = _rowstats(agg0, colia)
    rmax_s[0:_AH, :] = m0
    rmax_s[_AH:_H, :] = jnp.full((_H - _AH, 1), -1.0, jnp.float32)
    rcol_s[0:_AH, :] = c0
    rcol_s[_AH:_H, :] = jnp.zeros((_H - _AH, 1), jnp.int32)

    rowi16 = _iota((16, _W), 0)
    coli16 = _iota((16, _W), 1)
    rowi8 = _iota((8, _W), 0)
    rio = _iota((_H, 1), 0)
    big = jnp.int32(_H * _W)

    def body(i, res):
        rm = rmax_s[...]
        gmax = jnp.max(rm)
        r = jnp.min(jnp.where(rm == gmax, rio, jnp.int32(_H)))
        c = jnp.min(jnp.where(rio == r, rcol_s[...], jnp.int32(_W)))

        rs = pl.multiple_of(jnp.minimum((r // 8) * 8, _H - 16), 8)
        tile = hm_s[pl.ds(rs, 16), :]
        inwin = ((rowi16 >= r - rs) & (rowi16 < r - rs + _R)
                 & (coli16 >= c) & (coli16 < c + _R))
        mval = jnp.max(jnp.where(inwin, tile, -1.0))
        conf = jnp.sum(jnp.where(inwin, tile, 0.0))

        cmask = hmax_s[...] >= mval

        def scan_cond(st):
            fi, cursor = st
            return (fi == big) & (cursor < _H)

        def scan_step(st):
            fi, cursor = st
            nrow = jnp.min(jnp.where(cmask & (rio >= cursor), rio,
                                     jnp.int32(_H)))
            nrow_c = jnp.minimum(nrow, jnp.int32(_H - 1))
            b8 = pl.multiple_of(jnp.minimum((nrow_c // 8) * 8, _H - 8), 8)
            blk = hm_s[pl.ds(b8, 8), :]
            rowabs = b8 + rowi8
            hit = (blk == mval) & (rowabs >= nrow)
            fi2 = jnp.min(jnp.where(hit, rowabs * _W + coli16[0:8], big))
            fi2 = jnp.where(nrow < _H, fi2, big)
            return fi2, b8 + 8

        fi2, _ = jax.lax.while_loop(scan_cond, scan_step,
                                    (big, jnp.int32(0)))
        rh = fi2 // _W
        ch = jax.lax.rem(fi2, _W)

        ztile = jnp.where(inwin, 0.0, tile)
        hm_s[pl.ds(rs, 16), :] = ztile
        h_s[pl.ds(rs, 16), :] = _hsum(ztile)
        hmax_s[pl.ds(rs, 16), :] = jnp.max(ztile, axis=1, keepdims=True)

        rs3 = pl.multiple_of(
            jnp.minimum(jnp.maximum(((r - (_R - 1)) // 8) * 8, 0), _H - 16), 8)
        h32 = h_s[pl.ds(rs3, 32), :]
        acc = h32[0:16]
        for k in range(1, _R):
            acc = acc + h32[k:k + 16]
        rowabs16 = rs3 + rowi16
        aggblk = jnp.where((rowabs16 < _AH) & (coli16 < _AH),
                           acc / float(_R * _R), -1.0)
        mb, cb = _rowstats(aggblk, coli16)
        rmax_s[pl.ds(rs3, 16), :] = mb
        rcol_s[pl.ds(rs3, 16), :] = cb

        sel = _iota((8, 128), 0) == i
        coli8 = _iota((8, 128), 1)
        res = jnp.where(sel & (coli8 == 0), rh.astype(jnp.float32), res)
        res = jnp.where(sel & (coli8 == 1), ch.astype(jnp.float32), res)
        res = jnp.where(sel & (coli8 == 2), conf, res)
        return res

    res = jax.lax.fori_loop(0, _N_TARGETS, body,
                            jnp.zeros((8, 128), jnp.float32))
    out_ref[0] = res


def kernel(heatmap):
    hm = heatmap[:, 0]
    b = hm.shape[0]
    out = pl.pallas_call(
        _peaks_kernel,
        grid=(b,),
        in_specs=[pl.BlockSpec((1, _H, _W), lambda i: (i, 0, 0))],
        out_specs=pl.BlockSpec((1, 8, 128), lambda i: (i, 0, 0)),
        out_shape=jax.ShapeDtypeStruct((b, 8, 128), jnp.float32),
        scratch_shapes=[
            pltpu.VMEM((_H, _W), jnp.float32),
            pltpu.VMEM((_HP, _W), jnp.float32),
            pltpu.VMEM((_H, 1), jnp.float32),
            pltpu.VMEM((_H, 1), jnp.int32),
            pltpu.VMEM((_H, 1), jnp.float32),
        ],
        compiler_params=pltpu.CompilerParams(
            dimension_semantics=("parallel",),
        ),
    )(hm)
    end_points = out[:, :_N_TARGETS, 0:2]
    confidences = out[:, :_N_TARGETS, 2]
    return end_points, confidences


# speedup vs baseline: 8.5415x; 7.2217x over previous
import jax
import jax.numpy as jnp
from jax.experimental import pallas as pl
from jax.experimental.pallas import tpu as pltpu

_N_TARGETS = 6
_R = 5
_H = 512
_W = 512
_AH = _H - _R + 1
_HP = _H + 32


def _iota(shape, dim):
    return jax.lax.broadcasted_iota(jnp.int32, shape, dim)


def _hsum(t):
    acc = t
    n = t.shape[0]
    for k in range(1, _R):
        acc = acc + jnp.concatenate(
            [t[:, k:], jnp.zeros((n, k), jnp.float32)], axis=1)
    return acc


def _rowstats(aggblk, colia):
    m = jnp.max(aggblk, axis=1, keepdims=True)
    cc = jnp.min(jnp.where(aggblk == m, colia, jnp.int32(_W)),
                 axis=1, keepdims=True)
    return m, cc


def _peaks_kernel(hm_ref, out_ref, hm_s, h_s, rmax_s, rcol_s, hmax_s):
    hm0 = hm_ref[0]
    hm_s[...] = hm0
    hmax_s[...] = jnp.max(hm0, axis=1, keepdims=True)

    h0 = _hsum(hm0)
    h_s[0:_H, :] = h0
    h_s[_H:_HP, :] = jnp.zeros((_HP - _H, _W), jnp.float32)

    v = h0[0:_AH, :]
    for k in range(1, _R):
        v = v + h0[k:k + _AH, :]
    colia = _iota((_AH, _W), 1)
    agg0 = jnp.where(colia < _AH, v / float(_R * _R), -1.0)
    m0, c0 = _rowstats(agg0, colia)
    rmax_s[0:_AH, :] = m0
    rmax_s[_AH:_H, :] = jnp.full((_H - _AH, 1), -1.0, jnp.float32)
    rcol_s[0:_AH, :] = c0
    rcol_s[_AH:_H, :] = jnp.zeros((_H - _AH, 1), jnp.int32)

    flat = _iota((_H, _W), 0) * _W + _iota((_H, _W), 1)
    big = jnp.int32(_H * _W)
    rowi16 = _iota((16, _W), 0)
    coli16 = _iota((16, _W), 1)
    rio = _iota((_H, 1), 0)

    def body(i, res):
        rm = rmax_s[...]
        gmax = jnp.max(rm)
        r = jnp.min(jnp.where(rm == gmax, rio, jnp.int32(_H)))
        c = jnp.min(jnp.where(rio == r, rcol_s[...], jnp.int32(_W)))

        rs = pl.multiple_of(jnp.minimum((r // 8) * 8, _H - 16), 8)
        tile = hm_s[pl.ds(rs, 16), :]
        inwin = ((rowi16 >= r - rs) & (rowi16 < r - rs + _R)
                 & (coli16 >= c) & (coli16 < c + _R))
        mval = jnp.max(jnp.where(inwin, tile, -1.0))
        conf = jnp.sum(jnp.where(inwin, tile, 0.0))

        rf = jnp.min(jnp.where(hmax_s[...] >= mval, rio, jnp.int32(_H)))
        bs = pl.multiple_of(jnp.minimum((rf // 8) * 8, _H - 16), 8)
        blk = hm_s[pl.ds(bs, 16), :]
        rowabs16 = bs + rowi16
        fi2 = jnp.min(jnp.where((blk == mval) & (rowabs16 >= rf),
                                rowabs16 * _W + coli16, big))

        def full_scan(_):
            return jnp.min(jnp.where(hm_s[...] == mval, flat, big))
        fi2 = jax.lax.cond(fi2 == big, full_scan, lambda _: fi2, 0)
        rh = fi2 // _W
        ch = jax.lax.rem(fi2, _W)

        ztile = jnp.where(inwin, 0.0, tile)
        hm_s[pl.ds(rs, 16), :] = ztile
        h_s[pl.ds(rs, 16), :] = _hsum(ztile)
        hmax_s[pl.ds(rs, 16), :] = jnp.max(ztile, axis=1, keepdims=True)

        rs3 = pl.multiple_of(
            jnp.minimum(jnp.maximum(((r - (_R - 1)) // 8) * 8, 0), _H - 16), 8)
        h32 = h_s[pl.ds(rs3, 32), :]
        acc = h32[0:16]
        for k in range(1, _R):
            acc = acc + h32[k:k + 16]
        rowabs = rs3 + rowi16
        aggblk = jnp.where((rowabs < _AH) & (coli16 < _AH),
                           acc / float(_R * _R), -1.0)
        mb, cb = _rowstats(aggblk, coli16)
        rmax_s[pl.ds(rs3, 16), :] = mb
        rcol_s[pl.ds(rs3, 16), :] = cb

        sel = _iota((8, 128), 0) == i
        coli8 = _iota((8, 128), 1)
        res = jnp.where(sel & (coli8 == 0), rh.astype(jnp.float32), res)
        res = jnp.where(sel & (coli8 == 1), ch.astype(jnp.float32), res)
        res = jnp.where(sel & (coli8 == 2), conf, res)
        return res

    res = jax.lax.fori_loop(0, _N_TARGETS, body,
                            jnp.zeros((8, 128), jnp.float32))
    out_ref[0] = res


def kernel(heatmap):
    hm = heatmap[:, 0]
    b = hm.shape[0]
    out = pl.pallas_call(
        _peaks_kernel,
        grid=(b,),
        in_specs=[pl.BlockSpec((1, _H, _W), lambda i: (i, 0, 0))],
        out_specs=pl.BlockSpec((1, 8, 128), lambda i: (i, 0, 0)),
        out_shape=jax.ShapeDtypeStruct((b, 8, 128), jnp.float32),
        scratch_shapes=[
            pltpu.VMEM((_H, _W), jnp.float32),
            pltpu.VMEM((_HP, _W), jnp.float32),
            pltpu.VMEM((_H, 1), jnp.float32),
            pltpu.VMEM((_H, 1), jnp.int32),
            pltpu.VMEM((_H, 1), jnp.float32),
        ],
        compiler_params=pltpu.CompilerParams(
            dimension_semantics=("parallel",),
        ),
    )(hm)
    end_points = out[:, :_N_TARGETS, 0:2]
    confidences = out[:, :_N_TARGETS, 2]
    return end_points, confidences


# speedup vs baseline: 13.4087x; 1.5698x over previous
import jax
import jax.numpy as jnp
from jax.experimental import pallas as pl
from jax.experimental.pallas import tpu as pltpu

_N_TARGETS = 6
_R = 5
_H = 512
_W = 512
_AH = _H - _R + 1
_HP = _H + 32


def _iota(shape, dim):
    return jax.lax.broadcasted_iota(jnp.int32, shape, dim)


def _hsum(t):
    acc = t
    n = t.shape[0]
    for k in range(1, _R):
        acc = acc + jnp.concatenate(
            [t[:, k:], jnp.zeros((n, k), jnp.float32)], axis=1)
    return acc


def _rowstats(aggblk, colia):
    m = jnp.max(aggblk, axis=1, keepdims=True)
    cc = jnp.min(jnp.where(aggblk == m, colia, jnp.int32(_W)),
                 axis=1, keepdims=True)
    return m, cc


_flat = None


def _peaks_kernel(hm_ref, out_ref, hm_s, h_s, rmax_s, rcol_s):
    colia = _iota((_AH, _W), 1)
    flat = _iota((_H, _W), 0) * _W + _iota((_H, _W), 1)
    big = jnp.int32(_H * _W)
    rowi16 = _iota((16, _W), 0)
    coli16 = _iota((16, _W), 1)
    rio = _iota((_H, 1), 0)

    def init(q):
        hm0 = hm_ref[q]
        hm_s[q * _H:(q + 1) * _H, :] = hm0
        h0 = _hsum(hm0)
        h_s[q * _HP:q * _HP + _H, :] = h0
        h_s[q * _HP + _H:(q + 1) * _HP, :] = jnp.zeros(
            (_HP - _H, _W), jnp.float32)
        v = h0[0:_AH, :]
        for k in range(1, _R):
            v = v + h0[k:k + _AH, :]
        agg0 = jnp.where(colia < _AH, v / float(_R * _R), -1.0)
        m0, c0 = _rowstats(agg0, colia)
        rmax_s[q * _H:q * _H + _AH, :] = m0
        rmax_s[q * _H + _AH:(q + 1) * _H, :] = jnp.full(
            (_H - _AH, 1), -1.0, jnp.float32)
        rcol_s[q * _H:q * _H + _AH, :] = c0
        rcol_s[q * _H + _AH:(q + 1) * _H, :] = jnp.zeros(
            (_H - _AH, 1), jnp.int32)

    init(0)
    init(1)

    def one(q, i, res):
        hb = q * _H
        rm = rmax_s[hb:hb + _H, :]
        gmax = jnp.max(rm)
        r = jnp.min(jnp.where(rm == gmax, rio, jnp.int32(_H)))
        c = jnp.min(jnp.where(rio == r, rcol_s[hb:hb + _H, :],
                              jnp.int32(_W)))

        rs = pl.multiple_of(hb + jnp.minimum((r // 8) * 8, _H - 16), 8)
        tile = hm_s[pl.ds(rs, 16), :]
        rr = r + hb - rs
        inwin = ((rowi16 >= rr) & (rowi16 < rr + _R)
                 & (coli16 >= c) & (coli16 < c + _R))
        mval = jnp.max(jnp.where(inwin, tile, -1.0))
        conf = jnp.sum(jnp.where(inwin, tile, 0.0))

        hm = hm_s[hb:hb + _H, :]
        fi2 = jnp.min(jnp.where(hm == mval, flat, big))
        rh = fi2 // _W
        ch = jax.lax.rem(fi2, _W)

        ztile = jnp.where(inwin, 0.0, tile)
        hm_s[pl.ds(rs, 16), :] = ztile
        h_s[pl.ds(pl.multiple_of(rs + q * (_HP - _H), 8), 16), :] = (
            _hsum(ztile))

        rs3 = pl.multiple_of(
            q * _HP + jnp.minimum(
                jnp.maximum(((r - (_R - 1)) // 8) * 8, 0), _H - 16), 8)
        h32 = h_s[pl.ds(rs3, 32), :]
        acc = h32[0:16]
        for k in range(1, _R):
            acc = acc + h32[k:k + 16]
        rowabs16 = (rs3 - q * _HP) + rowi16
        aggblk = jnp.where((rowabs16 < _AH) & (coli16 < _AH),
                           acc / float(_R * _R), -1.0)
        mb, cb = _rowstats(aggblk, coli16)
        rmb = pl.multiple_of(rs3 - q * _HP + hb, 8)
        rmax_s[pl.ds(rmb, 16), :] = mb
        rcol_s[pl.ds(rmb, 16), :] = cb

        sel = _iota((8, 128), 0) == i
        coli8 = _iota((8, 128), 1)
        res = jnp.where(sel & (coli8 == 0), rh.astype(jnp.float32), res)
        res = jnp.where(sel & (coli8 == 1), ch.astype(jnp.float32), res)
        res = jnp.where(sel & (coli8 == 2), conf, res)
        return res

    def body(i, carry):
        res0, res1 = carry
        res0 = one(0, i, res0)
        res1 = one(1, i, res1)
        return res0, res1

    res0, res1 = jax.lax.fori_loop(
        0, _N_TARGETS, body,
        (jnp.zeros((8, 128), jnp.float32), jnp.zeros((8, 128), jnp.float32)))
    out_ref[0] = res0
    out_ref[1] = res1


def kernel(heatmap):
    hm = heatmap[:, 0]
    b = hm.shape[0]
    out = pl.pallas_call(
        _peaks_kernel,
        grid=(b // 2,),
        in_specs=[pl.BlockSpec((2, _H, _W), lambda i: (i, 0, 0))],
        out_specs=pl.BlockSpec((2, 8, 128), lambda i: (i, 0, 0)),
        out_shape=jax.ShapeDtypeStruct((b, 8, 128), jnp.float32),
        scratch_shapes=[
            pltpu.VMEM((2 * _H, _W), jnp.float32),
            pltpu.VMEM((2 * _HP, _W), jnp.float32),
            pltpu.VMEM((2 * _H, 1), jnp.float32),
            pltpu.VMEM((2 * _H, 1), jnp.int32),
        ],
        compiler_params=pltpu.CompilerParams(
            dimension_semantics=("parallel",),
        ),
    )(hm)
    end_points = out[:, :_N_TARGETS, 0:2]
    confidences = out[:, :_N_TARGETS, 2]
    return end_points, confidences


# speedup vs baseline: 13.6563x; 1.0185x over previous
import jax
import jax.numpy as jnp
from jax.experimental import pallas as pl
from jax.experimental.pallas import tpu as pltpu

_N_TARGETS = 6
_R = 5
_H = 512
_W = 512
_AH = _H - _R + 1
_HP = _H + 32
_NQ = 4


def _iota(shape, dim):
    return jax.lax.broadcasted_iota(jnp.int32, shape, dim)


def _hsum(t):
    acc = t
    n = t.shape[0]
    for k in range(1, _R):
        acc = acc + jnp.concatenate(
            [t[:, k:], jnp.zeros((n, k), jnp.float32)], axis=1)
    return acc


def _rowstats(aggblk, colia):
    m = jnp.max(aggblk, axis=1, keepdims=True)
    cc = jnp.min(jnp.where(aggblk == m, colia, jnp.int32(_W)),
                 axis=1, keepdims=True)
    return m, cc


_flat = None


def _peaks_kernel(hm_ref, out_ref, hm_s, h_s, rmax_s, rcol_s):
    colia = _iota((_AH, _W), 1)
    flat = _iota((_H, _W), 0) * _W + _iota((_H, _W), 1)
    big = jnp.int32(_H * _W)
    rowi16 = _iota((16, _W), 0)
    coli16 = _iota((16, _W), 1)
    rio = _iota((_H, 1), 0)

    def init(q):
        hm0 = hm_ref[q]
        hm_s[q * _H:(q + 1) * _H, :] = hm0
        h0 = _hsum(hm0)
        h_s[q * _HP:q * _HP + _H, :] = h0
        h_s[q * _HP + _H:(q + 1) * _HP, :] = jnp.zeros(
            (_HP - _H, _W), jnp.float32)
        v = h0[0:_AH, :]
        for k in range(1, _R):
            v = v + h0[k:k + _AH, :]
        agg0 = jnp.where(colia < _AH, v / float(_R * _R), -1.0)
        m0, c0 = _rowstats(agg0, colia)
        rmax_s[q * _H:q * _H + _AH, :] = m0
        rmax_s[q * _H + _AH:(q + 1) * _H, :] = jnp.full(
            (_H - _AH, 1), -1.0, jnp.float32)
        rcol_s[q * _H:q * _H + _AH, :] = c0
        rcol_s[q * _H + _AH:(q + 1) * _H, :] = jnp.zeros(
            (_H - _AH, 1), jnp.int32)

    for q in range(_NQ):
        init(q)

    def one(q, i, res):
        hb = q * _H
        rm = rmax_s[hb:hb + _H, :]
        gmax = jnp.max(rm)
        r = jnp.min(jnp.where(rm == gmax, rio, jnp.int32(_H)))
        c = jnp.min(jnp.where(rio == r, rcol_s[hb:hb + _H, :],
                              jnp.int32(_W)))

        rs = pl.multiple_of(hb + jnp.minimum((r // 8) * 8, _H - 16), 8)
        tile = hm_s[pl.ds(rs, 16), :]
        rr = r + hb - rs
        inwin = ((rowi16 >= rr) & (rowi16 < rr + _R)
                 & (coli16 >= c) & (coli16 < c + _R))
        mval = jnp.max(jnp.where(inwin, tile, -1.0))
        conf = jnp.sum(jnp.where(inwin, tile, 0.0))

        hm = hm_s[hb:hb + _H, :]
        fi2 = jnp.min(jnp.where(hm == mval, flat, big))
        rh = fi2 // _W
        ch = jax.lax.rem(fi2, _W)

        ztile = jnp.where(inwin, 0.0, tile)
        hm_s[pl.ds(rs, 16), :] = ztile
        h_s[pl.ds(pl.multiple_of(rs + q * (_HP - _H), 8), 16), :] = (
            _hsum(ztile))

        rs3 = pl.multiple_of(
            q * _HP + jnp.minimum(
                jnp.maximum(((r - (_R - 1)) // 8) * 8, 0), _H - 16), 8)
        h32 = h_s[pl.ds(rs3, 32), :]
        acc = h32[0:16]
        for k in range(1, _R):
            acc = acc + h32[k:k + 16]
        rowabs16 = (rs3 - q * _HP) + rowi16
        aggblk = jnp.where((rowabs16 < _AH) & (coli16 < _AH),
                           acc / float(_R * _R), -1.0)
        mb, cb = _rowstats(aggblk, coli16)
        rmb = pl.multiple_of(rs3 - q * _HP + hb, 8)
        rmax_s[pl.ds(rmb, 16), :] = mb
        rcol_s[pl.ds(rmb, 16), :] = cb

        sel = _iota((8, 128), 0) == i
        coli8 = _iota((8, 128), 1)
        res = jnp.where(sel & (coli8 == 0), rh.astype(jnp.float32), res)
        res = jnp.where(sel & (coli8 == 1), ch.astype(jnp.float32), res)
        res = jnp.where(sel & (coli8 == 2), conf, res)
        return res

    def body(i, carry):
        return tuple(one(q, i, carry[q]) for q in range(_NQ))

    res = jax.lax.fori_loop(
        0, _N_TARGETS, body,
        tuple(jnp.zeros((8, 128), jnp.float32) for _ in range(_NQ)))
    for q in range(_NQ):
        out_ref[q] = res[q]


def kernel(heatmap):
    hm = heatmap[:, 0]
    b = hm.shape[0]
    out = pl.pallas_call(
        _peaks_kernel,
        grid=(b // _NQ,),
        in_specs=[pl.BlockSpec((_NQ, _H, _W), lambda i: (i, 0, 0))],
        out_specs=pl.BlockSpec((_NQ, 8, 128), lambda i: (i, 0, 0)),
        out_shape=jax.ShapeDtypeStruct((b, 8, 128), jnp.float32),
        scratch_shapes=[
            pltpu.VMEM((_NQ * _H, _W), jnp.float32),
            pltpu.VMEM((_NQ * _HP, _W), jnp.float32),
            pltpu.VMEM((_NQ * _H, 1), jnp.float32),
            pltpu.VMEM((_NQ * _H, 1), jnp.int32),
        ],
        compiler_params=pltpu.CompilerParams(
            dimension_semantics=("parallel",),
        ),
    )(hm)
    end_points = out[:, :_N_TARGETS, 0:2]
    confidences = out[:, :_N_TARGETS, 2]
    return end_points, confidences
